# baseline (speedup 1.0000x reference)
import functools

import jax
import jax.numpy as jnp
from jax import lax
from jax.experimental import pallas as pl
from jax.experimental.pallas import tpu as pltpu
from jax.experimental.pallas import tpu_sc as plsc

VOCAB_R = 1000
DIM_E = 128
DIM_R = 64
RSTRIDE = 65
NC = 2
NS = 16
NW = NC * NS
HIST = 50


@functools.lru_cache(maxsize=None)
def _make_sc_kernel(b: int, hist: int):
  bpw = b // NW
  assert b % NW == 0 and bpw % 16 == 0
  mesh = plsc.VectorSubcoreMesh(core_axis_name="c", subcore_axis_name="s")

  @functools.partial(
      pl.kernel,
      mesh=mesh,
      out_type=(
          jax.ShapeDtypeStruct((hist, b, DIM_E), jnp.float32),
          jax.ShapeDtypeStruct((hist, DIM_R, b), jnp.float32),
      ),
      compiler_params=pltpu.CompilerParams(needs_layout_passes=False),
      scratch_types=[
          pltpu.VMEM((VOCAB_R * RSTRIDE,), jnp.float32),
          pltpu.VMEM((hist, bpw), jnp.int32),
          pltpu.VMEM((hist, bpw), jnp.int32),
          pltpu.VMEM((2, bpw, DIM_E), jnp.float32),
          pltpu.VMEM((2, DIM_R, bpw), jnp.float32),
          pltpu.SemaphoreType.DMA,
          pltpu.SemaphoreType.DMA,
          pltpu.SemaphoreType.DMA,
          pltpu.SemaphoreType.DMA,
          pltpu.SemaphoreType.DMA,
          pltpu.SemaphoreType.DMA,
      ],
  )
  def sc_kernel(etab, rtab, eidx_h, ridx_h, out_e, out_r,
                rtab_v, eidx_v, ridx_v, erows, rrows,
                ge0, ge1, we0, we1, wr0, wr1):
    wid = lax.axis_index("s") * NC + lax.axis_index("c")
    base = wid * bpw
    ge = (ge0, ge1)
    we = (we0, we1)
    wr = (wr0, wr1)

    pltpu.sync_copy(rtab, rtab_v)
    pltpu.sync_copy(eidx_h.at[:, pl.ds(base, bpw)], eidx_v)
    pltpu.sync_copy(ridx_h.at[:, pl.ds(base, bpw)], ridx_v)

    def issue_gather(h, s):
      pltpu.async_copy(etab.at[eidx_v.at[h]], erows.at[s], ge[s])

    def wait_gather(h, s):
      pltpu.make_async_copy(etab.at[eidx_v.at[h]], erows.at[s], ge[s]).wait()

    def ent_wb(h, s):
      pltpu.async_copy(erows.at[s], out_e.at[h, pl.ds(base, bpw)], we[s])

    def wait_ent_wb(h, s):
      pltpu.make_async_copy(erows.at[s], out_e.at[h, pl.ds(base, bpw)],
                            we[s]).wait()

    def rel_wb(h, s):
      pltpu.async_copy(rrows.at[s], out_r.at[h, :, pl.ds(base, bpw)], wr[s])

    def wait_rel_wb(h, s):
      pltpu.make_async_copy(rrows.at[s], out_r.at[h, :, pl.ds(base, bpw)],
                            wr[s]).wait()

    def fill_rel(h, s):
      ivs = [ridx_v[h, pl.ds(16 * j, 16)] * RSTRIDE for j in range(bpw // 16)]
      for c0 in range(0, DIM_R, 2):
        vals = [plsc.load_gather(rtab_v, [ivs[j] + (c0 + k)])
                for j in range(bpw // 16) for k in range(2)]
        for j in range(bpw // 16):
          for k in range(2):
            rrows[s, c0 + k, pl.ds(16 * j, 16)] = vals[2 * j + k]

    def stage_b(h, s):
      @pl.when(h >= 2)
      def _():
        wait_rel_wb(h - 2, s)
      fill_rel(h, s)
      rel_wb(h, s)
      wait_gather(h, s)
      ent_wb(h, s)

    @pl.loop(0, hist, step=2)
    def _(h0):
      for dp in (0, 1):
        h = h0 + dp
        s = dp

        @pl.when(h >= 2)
        def _():
          wait_ent_wb(h - 2, s)
        issue_gather(h, s)

        @pl.when(h >= 1)
        def _():
          stage_b(h - 1, 1 - s)

    stage_b(hist - 1, 1)
    wait_ent_wb(hist - 2, 0)
    wait_ent_wb(hist - 1, 1)
    wait_rel_wb(hist - 2, 0)
    wait_rel_wb(hist - 1, 1)

  return sc_kernel


def kernel(table_entities, table_relations, entities_idx, relations_idx):
  b, h = entities_idx.shape
  eidx = entities_idx.astype(jnp.int32).T
  ridx = relations_idx.astype(jnp.int32).T
  rtab = jnp.pad(table_relations, ((0, 0), (0, RSTRIDE - DIM_R))).reshape(VOCAB_R * RSTRIDE)
  out_e, out_r = _make_sc_kernel(b, h)(table_entities, rtab, eidx, ridx)
  return (out_e.transpose(1, 0, 2), out_r.transpose(2, 0, 1))
